# baseline (speedup 1.0000x reference)
import functools

import jax
import jax.numpy as jnp
from jax.experimental import pallas as pl

_SIGMA = 0.5
_INV = 1.0 / (_SIGMA * (2.0 ** 0.5))
_ROWS = 4096
_COLS = 1024
_CR = 256


def _body(x_ref, mu_ref, nz_ref, out_ref, acc_ref):
    mu = mu_ref[...]
    gate = jnp.clip(mu[None, :, :] + _SIGMA * nz_ref[...], 0.0, 1.0)
    out_ref[...] = x_ref[...] * gate
    p = 0.5 * (1.0 + jax.lax.erf(mu * _INV))
    s = jnp.sum(p).reshape(1, 1)

    @pl.when(pl.program_id(0) == 0)
    def _init():
        acc_ref[...] = s

    @pl.when(pl.program_id(0) != 0)
    def _accum():
        acc_ref[...] += s


@jax.jit
def kernel(input_tensor, mu, noise):
    b, r, c = input_tensor.shape
    mu2 = mu.reshape(r, c)
    nz = noise.reshape(b, r, c)
    grid = r // _CR
    gated, acc = pl.pallas_call(
        _body,
        grid=(grid,),
        in_specs=[
            pl.BlockSpec((b, _CR, c), lambda i: (0, i, 0)),
            pl.BlockSpec((_CR, c), lambda i: (i, 0)),
            pl.BlockSpec((b, _CR, c), lambda i: (0, i, 0)),
        ],
        out_specs=[
            pl.BlockSpec((b, _CR, c), lambda i: (0, i, 0)),
            pl.BlockSpec((1, 1), lambda i: (0, 0)),
        ],
        out_shape=[
            jax.ShapeDtypeStruct((b, r, c), jnp.float32),
            jax.ShapeDtypeStruct((1, 1), jnp.float32),
        ],
    )(input_tensor, mu2, nz)
    return gated, acc[0, 0]
